# baseline (speedup 1.0000x reference)
import functools

import jax
import jax.numpy as jnp
from jax.experimental import pallas as pl
from jax.experimental.pallas import tpu as pltpu

_K = 4


def _gelu(x):
    return 0.5 * x * (1.0 + jax.lax.erf(x * jnp.float32(0.7071067811865476)))


def _stage1_kernel(t_ref, bank_ref, we1_ref, be1_ref, wo1_ref, bo1_ref,
                   wf1_ref, one0_ref, one1_ref, one2_ref, one3_ref, idx_ref):
    t = t_ref[...]
    bank = bank_ref[...]
    fm = jnp.mean(bank, axis=1)
    we1 = we1_ref[...]
    be1 = be1_ref[...]
    wo1 = wo1_ref[...]
    bo1 = bo1_ref[...]
    wf1 = wf1_ref[...]

    c_dim = fm.shape[0]
    t2 = jnp.sum(t * t, axis=1, keepdims=True)
    m2 = jnp.sum(fm * fm, axis=1)
    cross = jax.lax.dot_general(t, fm, (((1,), (1,)), ((), ())),
                                preferred_element_type=jnp.float32)
    d2 = t2 + m2[None, :] - 2.0 * cross
    dist = jnp.sqrt(jnp.maximum(d2, 0.0))

    iota = jax.lax.broadcasted_iota(jnp.int32, dist.shape, 1)
    nearest = []
    dwork = dist
    for j in range(_K):
        minv = jnp.min(dwork, axis=1, keepdims=True)
        idxv = jnp.min(jnp.where(dwork <= minv, iota, c_dim), axis=1)
        onehot = iota == idxv[:, None]
        idx_ref[:, j:j + 1] = idxv[:, None]
        dwork = jnp.where(onehot, jnp.float32(jnp.inf), dwork)
        nearest.append(
            jax.lax.dot_general(onehot.astype(jnp.float32), fm,
                                (((1,), (0,)), ((), ())),
                                preferred_element_type=jnp.float32))

    ef1 = []
    w1 = []
    for j in range(_K):
        e = _gelu(jnp.dot(nearest[j] - t, we1,
                          preferred_element_type=jnp.float32) + be1)
        ef1.append(e)
        w1.append(jnp.dot(e, wf1, preferred_element_type=jnp.float32))
    m = jnp.maximum(jnp.maximum(w1[0], w1[1]), jnp.maximum(w1[2], w1[3]))
    exps = [jnp.exp(w - m) for w in w1]
    ssum = exps[0] + exps[1] + exps[2] + exps[3]
    outs = (one0_ref, one1_ref, one2_ref, one3_ref)
    for j in range(_K):
        efm = ef1[j] * (exps[j] / ssum)
        off = jnp.tanh(jnp.dot(t + efm, wo1,
                               preferred_element_type=jnp.float32) + bo1)
        outs[j][...] = (1.0 + off) * t


def _stage2_kernel(idx_ref, one_ref, bank_ref, we2_ref, be2_ref, wo2_ref,
                   bo2_ref, wf2_ref, out_ref, pre_ref, *, b_dim, s_dim,
                   rows_per_iter):
    we2 = we2_ref[...]
    wo2 = wo2_ref[...]
    bo2 = bo2_ref[...]
    wf2 = wf2_ref[...]
    r = rows_per_iter
    n_chunk = r * _K
    n_rows = n_chunk * s_dim

    pre_ref[...] = jnp.dot(bank_ref[...], we2,
                           preferred_element_type=jnp.float32) + be2_ref[...]

    i0 = jax.lax.broadcasted_iota(jnp.int32, (n_chunk, n_rows), 0)
    i1 = jax.lax.broadcasted_iota(jnp.int32, (n_chunk, n_rows), 1)
    seg = ((i1 >= i0 * s_dim) & (i1 < (i0 + 1) * s_dim)).astype(jnp.float32)

    def body(it, carry):
        base = it * r
        pre_list = []
        ones_small = []
        for rr in range(r):
            row = base + rr
            ones_small.append(one_ref[pl.ds(row * _K, _K), :])
            for j in range(_K):
                c = idx_ref[row, j]
                pre_list.append(pre_ref[pl.ds(c * s_dim, s_dim), :])
        pre = jnp.concatenate(pre_list, axis=0)
        ones_cat = jnp.concatenate(ones_small, axis=0)

        onew = jnp.dot(ones_cat, we2, preferred_element_type=jnp.float32)
        x3 = pre.reshape(n_chunk, s_dim, -1) - onew[:, None, :]
        ef2 = _gelu(x3.reshape(n_rows, -1))
        w2 = jnp.dot(ef2, wf2, preferred_element_type=jnp.float32)
        w23 = w2.reshape(n_chunk, s_dim, -1)
        mq = jnp.max(w23, axis=1, keepdims=True)
        eq = jnp.exp(w23 - mq)
        sq = jnp.dot(seg, eq.reshape(n_rows, -1),
                     preferred_element_type=jnp.float32)
        rs = 1.0 / sq
        ef2m = (ef2.reshape(n_chunk, s_dim, -1) * eq) * rs[:, None, :]
        h = jnp.dot(ef2m.reshape(n_rows, -1), wo2,
                    preferred_element_type=jnp.float32)
        hbase = jnp.dot(ones_cat, wo2, preferred_element_type=jnp.float32) + bo2
        off2 = jnp.tanh(h.reshape(n_chunk, s_dim, -1) + hbase[:, None, :])
        red = jnp.dot(seg, off2.reshape(n_rows, -1),
                      preferred_element_type=jnp.float32)
        out_ref[pl.ds(base * _K, n_chunk), :] = ones_cat * (
            jnp.float32(s_dim) + red)
        return carry

    jax.lax.fori_loop(0, b_dim // r, body, 0)


def kernel(target_token, fea_bank, W_e1, b_e1, W_o1, b_o1, W_e2, b_e2,
           W_o2, b_o2, W_f1, W_f2):
    b_dim, d_dim = target_token.shape
    c_dim, s_dim, _ = fea_bank.shape

    be1 = b_e1.reshape(1, d_dim)
    bo1 = b_o1.reshape(1, d_dim)
    be2 = b_e2.reshape(1, d_dim)
    bo2 = b_o2.reshape(1, d_dim)

    out1 = [jax.ShapeDtypeStruct((b_dim, d_dim), jnp.float32)
            for _ in range(_K)]
    out1.append(jax.ShapeDtypeStruct((b_dim, _K), jnp.int32))
    ones_and_idx = pl.pallas_call(
        _stage1_kernel,
        out_shape=out1,
    )(target_token, fea_bank, W_e1, be1, W_o1, bo1, W_f1)
    one_parts = ones_and_idx[:_K]
    idx = ones_and_idx[_K]

    one_stage = jnp.stack(one_parts, axis=1)
    one2d = one_stage.reshape(b_dim * _K, d_dim)
    bank2d = fea_bank.reshape(c_dim * s_dim, d_dim)

    rows_per_iter = 2
    sec2d = pl.pallas_call(
        functools.partial(_stage2_kernel, b_dim=b_dim, s_dim=s_dim,
                          rows_per_iter=rows_per_iter),
        in_specs=[
            pl.BlockSpec(memory_space=pltpu.MemorySpace.SMEM),
            pl.BlockSpec(memory_space=pltpu.MemorySpace.VMEM),
            pl.BlockSpec(memory_space=pltpu.MemorySpace.VMEM),
            pl.BlockSpec(memory_space=pltpu.MemorySpace.VMEM),
            pl.BlockSpec(memory_space=pltpu.MemorySpace.VMEM),
            pl.BlockSpec(memory_space=pltpu.MemorySpace.VMEM),
            pl.BlockSpec(memory_space=pltpu.MemorySpace.VMEM),
            pl.BlockSpec(memory_space=pltpu.MemorySpace.VMEM),
        ],
        out_shape=jax.ShapeDtypeStruct((b_dim * _K, d_dim), jnp.float32),
        scratch_shapes=[pltpu.VMEM((c_dim * s_dim, d_dim), jnp.float32)],
    )(idx, one2d, bank2d, W_e2, be2, W_o2, bo2, W_f2)
    second_stage = sec2d.reshape(b_dim, _K, d_dim)
    return (one_stage, second_stage)


# speedup vs baseline: 1.3456x; 1.3456x over previous
import functools

import jax
import jax.numpy as jnp
from jax.experimental import pallas as pl
from jax.experimental.pallas import tpu as pltpu

_K = 4


def _gelu(x):
    return 0.5 * x * (1.0 + jax.lax.erf(x * jnp.float32(0.7071067811865476)))


def _stage1_kernel(t_ref, bank_ref, we1_ref, be1_ref, wo1_ref, bo1_ref,
                   wf1_ref, one0_ref, one1_ref, one2_ref, one3_ref, idx_ref):
    t = t_ref[...]
    bank = bank_ref[...]
    fm = jnp.mean(bank, axis=1)
    we1 = we1_ref[...]
    be1 = be1_ref[...]
    wo1 = wo1_ref[...]
    bo1 = bo1_ref[...]
    wf1 = wf1_ref[...]

    c_dim = fm.shape[0]
    t2 = jnp.sum(t * t, axis=1, keepdims=True)
    m2 = jnp.sum(fm * fm, axis=1)
    cross = jax.lax.dot_general(t, fm, (((1,), (1,)), ((), ())),
                                preferred_element_type=jnp.float32)
    d2 = t2 + m2[None, :] - 2.0 * cross
    dist = jnp.sqrt(jnp.maximum(d2, 0.0))

    iota = jax.lax.broadcasted_iota(jnp.int32, dist.shape, 1)
    nearest = []
    dwork = dist
    for j in range(_K):
        minv = jnp.min(dwork, axis=1, keepdims=True)
        idxv = jnp.min(jnp.where(dwork <= minv, iota, c_dim), axis=1)
        onehot = iota == idxv[:, None]
        idx_ref[:, j:j + 1] = idxv[:, None]
        dwork = jnp.where(onehot, jnp.float32(jnp.inf), dwork)
        nearest.append(
            jax.lax.dot_general(onehot.astype(jnp.float32), fm,
                                (((1,), (0,)), ((), ())),
                                preferred_element_type=jnp.float32))

    ef1 = []
    w1 = []
    for j in range(_K):
        e = _gelu(jnp.dot(nearest[j] - t, we1,
                          preferred_element_type=jnp.float32) + be1)
        ef1.append(e)
        w1.append(jnp.dot(e, wf1, preferred_element_type=jnp.float32))
    m = jnp.maximum(jnp.maximum(w1[0], w1[1]), jnp.maximum(w1[2], w1[3]))
    exps = [jnp.exp(w - m) for w in w1]
    ssum = exps[0] + exps[1] + exps[2] + exps[3]
    outs = (one0_ref, one1_ref, one2_ref, one3_ref)
    for j in range(_K):
        efm = ef1[j] * (exps[j] / ssum)
        off = jnp.tanh(jnp.dot(t + efm, wo1,
                               preferred_element_type=jnp.float32) + bo1)
        outs[j][...] = (1.0 + off) * t


def _stage2_kernel(idx_ref, one_ref, bank_ref, we2_ref, be2_ref, wo2_ref,
                   bo2_ref, wf2_ref, out_ref, pre_ref, *, b_dim, s_dim,
                   rows_per_iter):
    we2 = we2_ref[...]
    wo2 = wo2_ref[...]
    bo2 = bo2_ref[...]
    wf2 = wf2_ref[...]
    r = rows_per_iter
    n_chunk = r * _K
    n_rows = n_chunk * s_dim

    pre_ref[...] = jnp.dot(bank_ref[...], we2,
                           preferred_element_type=jnp.float32) + be2_ref[...]

    def body(it, carry):
        base = it * r
        pre_list = []
        ones_small = []
        for rr in range(r):
            row = base + rr
            ones_small.append(one_ref[pl.ds(row * _K, _K), :])
            for j in range(_K):
                c = idx_ref[row, j]
                pre_list.append(pre_ref[pl.ds(c * s_dim, s_dim), :])
        pre = jnp.concatenate(pre_list, axis=0)
        ones_cat = jnp.concatenate(ones_small, axis=0)

        onew = jnp.dot(ones_cat, we2, preferred_element_type=jnp.float32)
        x3 = pre.reshape(n_chunk, s_dim, -1) - onew[:, None, :]
        ef2 = _gelu(x3.reshape(n_rows, -1))
        w2 = jnp.dot(ef2, wf2, preferred_element_type=jnp.float32)
        w23 = w2.reshape(n_chunk, s_dim, -1)
        mq = jnp.max(w23, axis=1, keepdims=True)
        eq = jnp.exp(w23 - mq)
        sq = jnp.sum(eq, axis=1, keepdims=True)
        rs = 1.0 / sq
        ef2m = (ef2.reshape(n_chunk, s_dim, -1) * eq) * rs
        h = jnp.dot(ef2m.reshape(n_rows, -1), wo2,
                    preferred_element_type=jnp.float32)
        hbase = jnp.dot(ones_cat, wo2, preferred_element_type=jnp.float32) + bo2
        off2 = jnp.tanh(h.reshape(n_chunk, s_dim, -1) + hbase[:, None, :])
        red = jnp.sum(off2, axis=1)
        out_ref[pl.ds(base * _K, n_chunk), :] = ones_cat * (
            jnp.float32(s_dim) + red)
        return carry

    jax.lax.fori_loop(0, b_dim // r, body, 0)


def kernel(target_token, fea_bank, W_e1, b_e1, W_o1, b_o1, W_e2, b_e2,
           W_o2, b_o2, W_f1, W_f2):
    b_dim, d_dim = target_token.shape
    c_dim, s_dim, _ = fea_bank.shape

    be1 = b_e1.reshape(1, d_dim)
    bo1 = b_o1.reshape(1, d_dim)
    be2 = b_e2.reshape(1, d_dim)
    bo2 = b_o2.reshape(1, d_dim)

    out1 = [jax.ShapeDtypeStruct((b_dim, d_dim), jnp.float32)
            for _ in range(_K)]
    out1.append(jax.ShapeDtypeStruct((b_dim, _K), jnp.int32))
    ones_and_idx = pl.pallas_call(
        _stage1_kernel,
        out_shape=out1,
    )(target_token, fea_bank, W_e1, be1, W_o1, bo1, W_f1)
    one_parts = ones_and_idx[:_K]
    idx = ones_and_idx[_K]

    one_stage = jnp.stack(one_parts, axis=1)
    one2d = one_stage.reshape(b_dim * _K, d_dim)
    bank2d = fea_bank.reshape(c_dim * s_dim, d_dim)

    rows_per_iter = 2
    sec2d = pl.pallas_call(
        functools.partial(_stage2_kernel, b_dim=b_dim, s_dim=s_dim,
                          rows_per_iter=rows_per_iter),
        in_specs=[
            pl.BlockSpec(memory_space=pltpu.MemorySpace.SMEM),
            pl.BlockSpec(memory_space=pltpu.MemorySpace.VMEM),
            pl.BlockSpec(memory_space=pltpu.MemorySpace.VMEM),
            pl.BlockSpec(memory_space=pltpu.MemorySpace.VMEM),
            pl.BlockSpec(memory_space=pltpu.MemorySpace.VMEM),
            pl.BlockSpec(memory_space=pltpu.MemorySpace.VMEM),
            pl.BlockSpec(memory_space=pltpu.MemorySpace.VMEM),
            pl.BlockSpec(memory_space=pltpu.MemorySpace.VMEM),
        ],
        out_shape=jax.ShapeDtypeStruct((b_dim * _K, d_dim), jnp.float32),
        scratch_shapes=[pltpu.VMEM((c_dim * s_dim, d_dim), jnp.float32)],
    )(idx, one2d, bank2d, W_e2, be2, W_o2, bo2, W_f2)
    second_stage = sec2d.reshape(b_dim, _K, d_dim)
    return (one_stage, second_stage)


# speedup vs baseline: 1.4161x; 1.0523x over previous
import functools

import jax
import jax.numpy as jnp
from jax.experimental import pallas as pl
from jax.experimental.pallas import tpu as pltpu

_K = 4


def _tree_reduce_axis1(x, op):
    s = x.shape[1]
    while s > 1:
        h = s // 2
        x = op(x[:, :h, :], x[:, h:, :])
        s = h
    return x


def _gelu(x):
    return 0.5 * x * (1.0 + jax.lax.erf(x * jnp.float32(0.7071067811865476)))


def _stage1_kernel(t_ref, bank_ref, we1_ref, be1_ref, wo1_ref, bo1_ref,
                   wf1_ref, one0_ref, one1_ref, one2_ref, one3_ref, idx_ref):
    t = t_ref[...]
    bank = bank_ref[...]
    fm = jnp.mean(bank, axis=1)
    we1 = we1_ref[...]
    be1 = be1_ref[...]
    wo1 = wo1_ref[...]
    bo1 = bo1_ref[...]
    wf1 = wf1_ref[...]

    c_dim = fm.shape[0]
    t2 = jnp.sum(t * t, axis=1, keepdims=True)
    m2 = jnp.sum(fm * fm, axis=1)
    cross = jax.lax.dot_general(t, fm, (((1,), (1,)), ((), ())),
                                preferred_element_type=jnp.float32)
    d2 = t2 + m2[None, :] - 2.0 * cross
    dist = jnp.sqrt(jnp.maximum(d2, 0.0))

    iota = jax.lax.broadcasted_iota(jnp.int32, dist.shape, 1)
    nearest = []
    dwork = dist
    for j in range(_K):
        minv = jnp.min(dwork, axis=1, keepdims=True)
        idxv = jnp.min(jnp.where(dwork <= minv, iota, c_dim), axis=1)
        onehot = iota == idxv[:, None]
        idx_ref[:, j:j + 1] = idxv[:, None]
        dwork = jnp.where(onehot, jnp.float32(jnp.inf), dwork)
        nearest.append(
            jax.lax.dot_general(onehot.astype(jnp.float32), fm,
                                (((1,), (0,)), ((), ())),
                                preferred_element_type=jnp.float32))

    ef1 = []
    w1 = []
    for j in range(_K):
        e = _gelu(jnp.dot(nearest[j] - t, we1,
                          preferred_element_type=jnp.float32) + be1)
        ef1.append(e)
        w1.append(jnp.dot(e, wf1, preferred_element_type=jnp.float32))
    m = jnp.maximum(jnp.maximum(w1[0], w1[1]), jnp.maximum(w1[2], w1[3]))
    exps = [jnp.exp(w - m) for w in w1]
    ssum = exps[0] + exps[1] + exps[2] + exps[3]
    outs = (one0_ref, one1_ref, one2_ref, one3_ref)
    for j in range(_K):
        efm = ef1[j] * (exps[j] / ssum)
        off = jnp.tanh(jnp.dot(t + efm, wo1,
                               preferred_element_type=jnp.float32) + bo1)
        outs[j][...] = (1.0 + off) * t


def _stage2_kernel(idx_ref, one_ref, bank_ref, we2_ref, be2_ref, wo2_ref,
                   bo2_ref, wf2_ref, out_ref, pre_ref, *, b_dim, s_dim,
                   rows_per_iter):
    we2 = we2_ref[...]
    wo2 = wo2_ref[...]
    bo2 = bo2_ref[...]
    wf2 = wf2_ref[...]
    r = rows_per_iter
    n_chunk = r * _K
    n_rows = n_chunk * s_dim

    pre_ref[...] = jnp.dot(bank_ref[...], we2,
                           preferred_element_type=jnp.float32) + be2_ref[...]

    def body(it, carry):
        base = it * r
        pre_list = []
        ones_small = []
        for rr in range(r):
            row = base + rr
            ones_small.append(one_ref[pl.ds(row * _K, _K), :])
            for j in range(_K):
                c = idx_ref[row, j]
                pre_list.append(pre_ref[pl.ds(c * s_dim, s_dim), :])
        pre = jnp.concatenate(pre_list, axis=0)
        ones_cat = jnp.concatenate(ones_small, axis=0)

        onew = jnp.dot(ones_cat, we2, preferred_element_type=jnp.float32)
        x3 = pre.reshape(n_chunk, s_dim, -1) - onew[:, None, :]
        ef2 = _gelu(x3.reshape(n_rows, -1))
        w2 = jnp.dot(ef2, wf2, preferred_element_type=jnp.float32)
        w23 = w2.reshape(n_chunk, s_dim, -1)
        mq = _tree_reduce_axis1(w23, jnp.maximum)
        eq = jnp.exp(w23 - mq)
        sq = _tree_reduce_axis1(eq, jnp.add)
        rs = 1.0 / sq
        ef2m = (ef2.reshape(n_chunk, s_dim, -1) * eq) * rs
        h = jnp.dot(ef2m.reshape(n_rows, -1), wo2,
                    preferred_element_type=jnp.float32)
        hbase = jnp.dot(ones_cat, wo2, preferred_element_type=jnp.float32) + bo2
        off2 = jnp.tanh(h.reshape(n_chunk, s_dim, -1) + hbase[:, None, :])
        red = _tree_reduce_axis1(off2, jnp.add)[:, 0, :]
        out_ref[pl.ds(base * _K, n_chunk), :] = ones_cat * (
            jnp.float32(s_dim) + red)
        return carry

    jax.lax.fori_loop(0, b_dim // r, body, 0)


def kernel(target_token, fea_bank, W_e1, b_e1, W_o1, b_o1, W_e2, b_e2,
           W_o2, b_o2, W_f1, W_f2):
    b_dim, d_dim = target_token.shape
    c_dim, s_dim, _ = fea_bank.shape

    be1 = b_e1.reshape(1, d_dim)
    bo1 = b_o1.reshape(1, d_dim)
    be2 = b_e2.reshape(1, d_dim)
    bo2 = b_o2.reshape(1, d_dim)

    out1 = [jax.ShapeDtypeStruct((b_dim, d_dim), jnp.float32)
            for _ in range(_K)]
    out1.append(jax.ShapeDtypeStruct((b_dim, _K), jnp.int32))
    ones_and_idx = pl.pallas_call(
        _stage1_kernel,
        out_shape=out1,
    )(target_token, fea_bank, W_e1, be1, W_o1, bo1, W_f1)
    one_parts = ones_and_idx[:_K]
    idx = ones_and_idx[_K]

    one_stage = jnp.stack(one_parts, axis=1)
    one2d = one_stage.reshape(b_dim * _K, d_dim)
    bank2d = fea_bank.reshape(c_dim * s_dim, d_dim)

    rows_per_iter = 2
    sec2d = pl.pallas_call(
        functools.partial(_stage2_kernel, b_dim=b_dim, s_dim=s_dim,
                          rows_per_iter=rows_per_iter),
        in_specs=[
            pl.BlockSpec(memory_space=pltpu.MemorySpace.SMEM),
            pl.BlockSpec(memory_space=pltpu.MemorySpace.VMEM),
            pl.BlockSpec(memory_space=pltpu.MemorySpace.VMEM),
            pl.BlockSpec(memory_space=pltpu.MemorySpace.VMEM),
            pl.BlockSpec(memory_space=pltpu.MemorySpace.VMEM),
            pl.BlockSpec(memory_space=pltpu.MemorySpace.VMEM),
            pl.BlockSpec(memory_space=pltpu.MemorySpace.VMEM),
            pl.BlockSpec(memory_space=pltpu.MemorySpace.VMEM),
        ],
        out_shape=jax.ShapeDtypeStruct((b_dim * _K, d_dim), jnp.float32),
        scratch_shapes=[pltpu.VMEM((c_dim * s_dim, d_dim), jnp.float32)],
    )(idx, one2d, bank2d, W_e2, be2, W_o2, bo2, W_f2)
    second_stage = sec2d.reshape(b_dim, _K, d_dim)
    return (one_stage, second_stage)


# speedup vs baseline: 1.6323x; 1.1527x over previous
import functools

import jax
import jax.numpy as jnp
from jax.experimental import pallas as pl
from jax.experimental.pallas import tpu as pltpu

_K = 4


def _tree_reduce_axis1(x, op):
    s = x.shape[1]
    while s > 1:
        h = s // 2
        x = op(x[:, :h, :], x[:, h:, :])
        s = h
    return x


def _gelu(x):
    return 0.5 * x * (1.0 + jax.lax.erf(x * jnp.float32(0.7071067811865476)))


def _stage1_kernel(t_ref, bank_ref, we1_ref, be1_ref, wo1_ref, bo1_ref,
                   wf1_ref, one0_ref, one1_ref, one2_ref, one3_ref, idx_ref):
    t = t_ref[...]
    bank = bank_ref[...]
    fm = jnp.mean(bank, axis=1)
    we1 = we1_ref[...]
    be1 = be1_ref[...]
    wo1 = wo1_ref[...]
    bo1 = bo1_ref[...]
    wf1 = wf1_ref[...]

    c_dim = fm.shape[0]
    t2 = jnp.sum(t * t, axis=1, keepdims=True)
    m2 = jnp.sum(fm * fm, axis=1)
    cross = jax.lax.dot_general(t, fm, (((1,), (1,)), ((), ())),
                                preferred_element_type=jnp.float32)
    d2 = t2 + m2[None, :] - 2.0 * cross
    dist = jnp.sqrt(jnp.maximum(d2, 0.0))

    iota = jax.lax.broadcasted_iota(jnp.int32, dist.shape, 1)
    nearest = []
    dwork = dist
    for j in range(_K):
        minv = jnp.min(dwork, axis=1, keepdims=True)
        idxv = jnp.min(jnp.where(dwork <= minv, iota, c_dim), axis=1)
        onehot = iota == idxv[:, None]
        idx_ref[:, j:j + 1] = idxv[:, None]
        dwork = jnp.where(onehot, jnp.float32(jnp.inf), dwork)
        nearest.append(
            jax.lax.dot_general(onehot.astype(jnp.float32), fm,
                                (((1,), (0,)), ((), ())),
                                preferred_element_type=jnp.float32))

    ef1 = []
    w1 = []
    for j in range(_K):
        e = _gelu(jnp.dot(nearest[j] - t, we1,
                          preferred_element_type=jnp.float32) + be1)
        ef1.append(e)
        w1.append(jnp.dot(e, wf1, preferred_element_type=jnp.float32))
    m = jnp.maximum(jnp.maximum(w1[0], w1[1]), jnp.maximum(w1[2], w1[3]))
    exps = [jnp.exp(w - m) for w in w1]
    ssum = exps[0] + exps[1] + exps[2] + exps[3]
    outs = (one0_ref, one1_ref, one2_ref, one3_ref)
    for j in range(_K):
        efm = ef1[j] * (exps[j] / ssum)
        off = jnp.tanh(jnp.dot(t + efm, wo1,
                               preferred_element_type=jnp.float32) + bo1)
        outs[j][...] = (1.0 + off) * t


def _stage2_kernel(idx_ref, one_ref, bank_ref, we2_ref, be2_ref, wo2_ref,
                   bo2_ref, wf2_ref, out_ref, pre_ref, *, b_dim, s_dim,
                   rows_per_iter):
    we2 = we2_ref[...]
    wo2 = wo2_ref[...]
    bo2 = bo2_ref[...]
    wf2 = wf2_ref[...]
    r = rows_per_iter
    n_chunk = r * _K
    n_rows = n_chunk * s_dim

    pre_ref[...] = jnp.dot(bank_ref[...], we2,
                           preferred_element_type=jnp.float32) + be2_ref[...]

    def body(it, carry):
        base = it * r
        pre_list = []
        ones_small = []
        for rr in range(r):
            row = base + rr
            ones_small.append(one_ref[pl.ds(row * _K, _K), :])
            for j in range(_K):
                c = idx_ref[row, j]
                pre_list.append(pre_ref[pl.ds(c * s_dim, s_dim), :])
        pre = jnp.concatenate(pre_list, axis=0)
        ones_cat = jnp.concatenate(ones_small, axis=0)

        onew = jnp.dot(ones_cat, we2, preferred_element_type=jnp.float32)
        x3 = pre.reshape(n_chunk, s_dim, -1) - onew[:, None, :]
        ef2 = _gelu(x3.reshape(n_rows, -1))
        w2 = jnp.dot(ef2, wf2, preferred_element_type=jnp.float32)
        w23 = w2.reshape(n_chunk, s_dim, -1)
        mq = _tree_reduce_axis1(w23, jnp.maximum)
        eq = jnp.exp(w23 - mq)
        sq = _tree_reduce_axis1(eq, jnp.add)
        rs = 1.0 / sq
        ef2m = (ef2.reshape(n_chunk, s_dim, -1) * eq) * rs
        h = jnp.dot(ef2m.reshape(n_rows, -1), wo2,
                    preferred_element_type=jnp.float32)
        hbase = jnp.dot(ones_cat, wo2, preferred_element_type=jnp.float32) + bo2
        off2 = jnp.tanh(h.reshape(n_chunk, s_dim, -1) + hbase[:, None, :])
        red = _tree_reduce_axis1(off2, jnp.add)[:, 0, :]
        out_ref[pl.ds(base * _K, n_chunk), :] = ones_cat * (
            jnp.float32(s_dim) + red)
        return carry

    jax.lax.fori_loop(0, b_dim // r, body, 0)


def kernel(target_token, fea_bank, W_e1, b_e1, W_o1, b_o1, W_e2, b_e2,
           W_o2, b_o2, W_f1, W_f2):
    b_dim, d_dim = target_token.shape
    c_dim, s_dim, _ = fea_bank.shape

    be1 = b_e1.reshape(1, d_dim)
    bo1 = b_o1.reshape(1, d_dim)
    be2 = b_e2.reshape(1, d_dim)
    bo2 = b_o2.reshape(1, d_dim)

    out1 = [jax.ShapeDtypeStruct((b_dim, d_dim), jnp.float32)
            for _ in range(_K)]
    out1.append(jax.ShapeDtypeStruct((b_dim, _K), jnp.int32))
    ones_and_idx = pl.pallas_call(
        _stage1_kernel,
        out_shape=out1,
    )(target_token, fea_bank, W_e1, be1, W_o1, bo1, W_f1)
    one_parts = ones_and_idx[:_K]
    idx = ones_and_idx[_K]

    one_stage = jnp.stack(one_parts, axis=1)
    one2d = one_stage.reshape(b_dim * _K, d_dim)
    bank2d = fea_bank.reshape(c_dim * s_dim, d_dim)

    rows_per_iter = 4
    sec2d = pl.pallas_call(
        functools.partial(_stage2_kernel, b_dim=b_dim, s_dim=s_dim,
                          rows_per_iter=rows_per_iter),
        in_specs=[
            pl.BlockSpec(memory_space=pltpu.MemorySpace.SMEM),
            pl.BlockSpec(memory_space=pltpu.MemorySpace.VMEM),
            pl.BlockSpec(memory_space=pltpu.MemorySpace.VMEM),
            pl.BlockSpec(memory_space=pltpu.MemorySpace.VMEM),
            pl.BlockSpec(memory_space=pltpu.MemorySpace.VMEM),
            pl.BlockSpec(memory_space=pltpu.MemorySpace.VMEM),
            pl.BlockSpec(memory_space=pltpu.MemorySpace.VMEM),
            pl.BlockSpec(memory_space=pltpu.MemorySpace.VMEM),
        ],
        out_shape=jax.ShapeDtypeStruct((b_dim * _K, d_dim), jnp.float32),
        scratch_shapes=[pltpu.VMEM((c_dim * s_dim, d_dim), jnp.float32)],
    )(idx, one2d, bank2d, W_e2, be2, W_o2, bo2, W_f2)
    second_stage = sec2d.reshape(b_dim, _K, d_dim)
    return (one_stage, second_stage)


# speedup vs baseline: 1.6713x; 1.0239x over previous
import functools

import jax
import jax.numpy as jnp
from jax.experimental import pallas as pl
from jax.experimental.pallas import tpu as pltpu

_K = 4


def _tree_reduce_axis1(x, op):
    s = x.shape[1]
    while s > 1:
        h = s // 2
        x = op(x[:, :h, :], x[:, h:, :])
        s = h
    return x


def _gelu(x):
    return 0.5 * x * (1.0 + jax.lax.erf(x * jnp.float32(0.7071067811865476)))


def _stage1_kernel(t_ref, bank_ref, we1_ref, be1_ref, wo1_ref, bo1_ref,
                   wf1_ref, one0_ref, one1_ref, one2_ref, one3_ref, idx_ref):
    t = t_ref[...]
    bank = bank_ref[...]
    fm = jnp.mean(bank, axis=1)
    we1 = we1_ref[...]
    be1 = be1_ref[...]
    wo1 = wo1_ref[...]
    bo1 = bo1_ref[...]
    wf1 = wf1_ref[...]

    c_dim = fm.shape[0]
    t2 = jnp.sum(t * t, axis=1, keepdims=True)
    m2 = jnp.sum(fm * fm, axis=1)
    cross = jax.lax.dot_general(t, fm, (((1,), (1,)), ((), ())),
                                preferred_element_type=jnp.float32)
    d2 = t2 + m2[None, :] - 2.0 * cross
    dist = jnp.sqrt(jnp.maximum(d2, 0.0))

    iota = jax.lax.broadcasted_iota(jnp.int32, dist.shape, 1)
    nearest = []
    dwork = dist
    for j in range(_K):
        minv = jnp.min(dwork, axis=1, keepdims=True)
        idxv = jnp.min(jnp.where(dwork <= minv, iota, c_dim), axis=1)
        onehot = iota == idxv[:, None]
        idx_ref[:, j:j + 1] = idxv[:, None]
        dwork = jnp.where(onehot, jnp.float32(jnp.inf), dwork)
        nearest.append(
            jax.lax.dot_general(onehot.astype(jnp.float32), fm,
                                (((1,), (0,)), ((), ())),
                                preferred_element_type=jnp.float32))

    ef1 = []
    w1 = []
    for j in range(_K):
        e = _gelu(jnp.dot(nearest[j] - t, we1,
                          preferred_element_type=jnp.float32) + be1)
        ef1.append(e)
        w1.append(jnp.dot(e, wf1, preferred_element_type=jnp.float32))
    m = jnp.maximum(jnp.maximum(w1[0], w1[1]), jnp.maximum(w1[2], w1[3]))
    exps = [jnp.exp(w - m) for w in w1]
    ssum = exps[0] + exps[1] + exps[2] + exps[3]
    outs = (one0_ref, one1_ref, one2_ref, one3_ref)
    for j in range(_K):
        efm = ef1[j] * (exps[j] / ssum)
        off = jnp.tanh(jnp.dot(t + efm, wo1,
                               preferred_element_type=jnp.float32) + bo1)
        outs[j][...] = (1.0 + off) * t


def _stage2_kernel(idx_ref, one_ref, bank_ref, we2_ref, be2_ref, wo2_ref,
                   bo2_ref, wf2_ref, out_ref, pre_ref, *, b_dim, s_dim,
                   rows_per_iter):
    we2 = we2_ref[...]
    wo2 = wo2_ref[...]
    bo2 = bo2_ref[...]
    wf2 = wf2_ref[...]
    r = rows_per_iter
    n_chunk = r * _K
    n_rows = n_chunk * s_dim

    pre_ref[...] = jnp.dot(bank_ref[...], we2,
                           preferred_element_type=jnp.float32) + be2_ref[...]

    def body(it, carry):
        base = it * r
        pre_list = []
        ones_small = []
        for rr in range(r):
            row = base + rr
            ones_small.append(one_ref[pl.ds(row * _K, _K), :])
            for j in range(_K):
                c = idx_ref[row, j]
                pre_list.append(pre_ref[pl.ds(c * s_dim, s_dim), :])
        pre = jnp.concatenate(pre_list, axis=0)
        ones_cat = jnp.concatenate(ones_small, axis=0)

        onew = jnp.dot(ones_cat, we2, preferred_element_type=jnp.float32)
        x3 = pre.reshape(n_chunk, s_dim, -1) - onew[:, None, :]
        ef2 = _gelu(x3.reshape(n_rows, -1))
        w2 = jnp.dot(ef2, wf2, preferred_element_type=jnp.float32)
        w23 = w2.reshape(n_chunk, s_dim, -1)
        mq = _tree_reduce_axis1(w23, jnp.maximum)
        eq = jnp.exp(w23 - mq)
        sq = _tree_reduce_axis1(eq, jnp.add)
        rs = 1.0 / sq
        ef2m = (ef2.reshape(n_chunk, s_dim, -1) * eq) * rs
        h = jnp.dot(ef2m.reshape(n_rows, -1), wo2,
                    preferred_element_type=jnp.float32)
        hbase = jnp.dot(ones_cat, wo2, preferred_element_type=jnp.float32) + bo2
        off2 = jnp.tanh(h.reshape(n_chunk, s_dim, -1) + hbase[:, None, :])
        red = _tree_reduce_axis1(off2, jnp.add)[:, 0, :]
        out_ref[pl.ds(base * _K, n_chunk), :] = ones_cat * (
            jnp.float32(s_dim) + red)
        return carry

    jax.lax.fori_loop(0, b_dim // r, body, 0)


def kernel(target_token, fea_bank, W_e1, b_e1, W_o1, b_o1, W_e2, b_e2,
           W_o2, b_o2, W_f1, W_f2):
    b_dim, d_dim = target_token.shape
    c_dim, s_dim, _ = fea_bank.shape

    be1 = b_e1.reshape(1, d_dim)
    bo1 = b_o1.reshape(1, d_dim)
    be2 = b_e2.reshape(1, d_dim)
    bo2 = b_o2.reshape(1, d_dim)

    out1 = [jax.ShapeDtypeStruct((b_dim, d_dim), jnp.float32)
            for _ in range(_K)]
    out1.append(jax.ShapeDtypeStruct((b_dim, _K), jnp.int32))
    ones_and_idx = pl.pallas_call(
        _stage1_kernel,
        out_shape=out1,
    )(target_token, fea_bank, W_e1, be1, W_o1, bo1, W_f1)
    one_parts = ones_and_idx[:_K]
    idx = ones_and_idx[_K]

    one_stage = jnp.stack(one_parts, axis=1)
    one2d = one_stage.reshape(b_dim * _K, d_dim)
    bank2d = fea_bank.reshape(c_dim * s_dim, d_dim)

    rows_per_iter = 8
    sec2d = pl.pallas_call(
        functools.partial(_stage2_kernel, b_dim=b_dim, s_dim=s_dim,
                          rows_per_iter=rows_per_iter),
        in_specs=[
            pl.BlockSpec(memory_space=pltpu.MemorySpace.SMEM),
            pl.BlockSpec(memory_space=pltpu.MemorySpace.VMEM),
            pl.BlockSpec(memory_space=pltpu.MemorySpace.VMEM),
            pl.BlockSpec(memory_space=pltpu.MemorySpace.VMEM),
            pl.BlockSpec(memory_space=pltpu.MemorySpace.VMEM),
            pl.BlockSpec(memory_space=pltpu.MemorySpace.VMEM),
            pl.BlockSpec(memory_space=pltpu.MemorySpace.VMEM),
            pl.BlockSpec(memory_space=pltpu.MemorySpace.VMEM),
        ],
        out_shape=jax.ShapeDtypeStruct((b_dim * _K, d_dim), jnp.float32),
        scratch_shapes=[pltpu.VMEM((c_dim * s_dim, d_dim), jnp.float32)],
    )(idx, one2d, bank2d, W_e2, be2, W_o2, bo2, W_f2)
    second_stage = sec2d.reshape(b_dim, _K, d_dim)
    return (one_stage, second_stage)
